# baseline (speedup 1.0000x reference)
import functools

import jax
import jax.numpy as jnp
from jax import lax
from jax.experimental import pallas as pl
from jax.experimental.pallas import tpu as pltpu
from jax.experimental.pallas import tpu_sc as plsc

N = 50000
D = 128
K = 16
ED = 64
B = 256
NREL = 3
SL = B + NREL * B * K
SLP = SL + 8
NT = 32
NGRP = SL // K
GPT = (NGRP + NT - 1) // NT

BLK1 = 512
NPH = 49 * BLK1
NP = 2 * NPH
ZROW = 2 * (N - NPH) + 1

PERM64 = [32 * b + (i // 2 if i % 2 == 0 else 16 + i // 2)
          for b in (0, 1) for i in range(32)]


def _k1_body(fa_ref, fb_ref, w0_ref, b0_ref, a1_ref, a2_ref, h0_ref, w_ref):
    pid = pl.program_id(0)
    ha = jnp.dot(fa_ref[...], w0_ref[...], preferred_element_type=jnp.float32)
    hb = jnp.dot(fb_ref[...], w0_ref[...], preferred_element_type=jnp.float32)
    bias = b0_ref[...][0:1, :]
    rowb = NPH + pid * BLK1 + lax.broadcasted_iota(jnp.int32, (BLK1, ED), 0)
    hb = jnp.where(rowb < N, hb + bias, 0.0)
    h0_ref[...] = jnp.concatenate([ha + bias, hb], axis=1).astype(jnp.bfloat16)

    @pl.when(pid == 0)
    def _():
        rid1 = lax.broadcasted_iota(jnp.int32, (8, 2 * ED * 2), 0)
        a1 = a1_ref[...]
        v1 = rid1 < NREL
        m1 = jnp.max(jnp.where(v1, a1, -jnp.inf), axis=0, keepdims=True)
        e1 = jnp.where(v1, jnp.exp(a1 - m1), 0.0)
        w1 = e1 / jnp.sum(e1, axis=0, keepdims=True)
        a2 = a2_ref[...]
        v2 = (rid1 >= 4) & (rid1 < 4 + NREL)
        m2 = jnp.max(jnp.where(v2, a2, -jnp.inf), axis=0, keepdims=True)
        e2 = jnp.where(v2, jnp.exp(a2 - m2), 0.0)
        w2 = e2 / jnp.sum(e2, axis=0, keepdims=True)
        w_ref[...] = jnp.where(v1, w1, 0.0) + jnp.where(v2, w2, 0.0)


def _run_k1(feat_data, W0, b0, a1t8, a2t8):
    b0b = jnp.broadcast_to(b0[None, :], (8, ED))
    nb = NPH // BLK1
    return pl.pallas_call(
        _k1_body,
        grid=(nb,),
        in_specs=[
            pl.BlockSpec((BLK1, D), lambda i: (i, 0)),
            pl.BlockSpec((BLK1, D), lambda i: (i + NPH // BLK1, 0)),
            pl.BlockSpec((D, ED), lambda i: (0, 0)),
            pl.BlockSpec((8, ED), lambda i: (0, 0)),
            pl.BlockSpec((8, 2 * ED * 2), lambda i: (0, 0)),
            pl.BlockSpec((8, 2 * ED * 2), lambda i: (0, 0)),
        ],
        out_specs=[
            pl.BlockSpec((BLK1, 2 * ED), lambda i: (i, 0)),
            pl.BlockSpec((8, 2 * ED * 2), lambda i: (0, 0)),
        ],
        out_shape=[
            jax.ShapeDtypeStruct((NPH, 2 * ED), jnp.bfloat16),
            jax.ShapeDtypeStruct((8, 2 * ED * 2), jnp.float32),
        ],
    )(feat_data, feat_data, W0, b0b, a1t8, a2t8)


def _dedup(neigh, iota, T):
    plsc.store_scatter(T, [neigh], iota)
    g = plsc.load_gather(T, [neigh])
    win = g == iota
    cnt = plsc.all_reduce_population_count(win)
    cinv = 1.0 / cnt.astype(jnp.float32)
    return win, cinv


def _k2_body(h0_hbm, nodes2_hbm, adj2d_hbm, w1a_hbm, w1b_hbm, h1_hbm,
             sids2, idxbuf, w1a_v, w1b_v, adjbuf, selfbuf, nfbuf, idxnf,
             cinvbuf, outbuf, T, semA, semS, semN, semO):
    wid = lax.axis_index("s") * 2 + lax.axis_index("c")
    iota = lax.iota(jnp.int32, 16)

    pltpu.sync_copy(nodes2_hbm, sids2.at[pl.ds(0, 16)])
    dStage = []
    for r in range(NREL):
        def _mk_idx(q, _):
            idxbuf[pl.ds(r * 256 + q * 16, 16)] = sids2[q, :] + r * N
            return _
        lax.fori_loop(0, 16, _mk_idx, None)
        dStage.append(pltpu.async_copy(
            adj2d_hbm.at[idxbuf.at[pl.ds(r * 256, 256)]],
            sids2.at[pl.ds(16 + r * 256, 256)], semA
        ))
    pltpu.sync_copy(w1a_hbm, w1a_v)
    pltpu.sync_copy(w1b_hbm, w1b_v)
    for d in dStage:
        d.wait()

    def pairmap(n):
        return jnp.where(n < NPH, 2 * n, 2 * n - (2 * NPH - 1))

    def fire_group(gg, sl):
        ids_vec = sids2[gg, :]
        pltpu.async_copy(h0_hbm.at[pairmap(ids_vec)], selfbuf.at[sl], semS)
        for r in range(NREL):
            pltpu.async_copy(
                adj2d_hbm.at[ids_vec + r * N], adjbuf.at[sl, r], semA
            )

    @pl.when(wid < NGRP)
    def _():
        fire_group(wid, 0)

    def group_body(it, _):
        gg = wid + it * NT
        sl = it % 2

        @pl.when(gg < NGRP)
        def _():
            for r in range(NREL):
                pltpu.make_async_copy(
                    adj2d_hbm.at[iota], adjbuf.at[sl, r], semA
                ).wait()

            def fire_half(h):
                def prep_row(j2, _):
                    for r in range(NREL):
                        neigh = adjbuf[sl, r, h * 8 + j2, :]
                        win, cinv = _dedup(neigh, iota, T)
                        idx2 = jnp.where(win, pairmap(neigh),
                                         jnp.full((16,), ZROW, jnp.int32))
                        slot = j2 * NREL + r
                        idxnf[h, pl.ds(slot * 16, 16)] = idx2
                        cinvbuf[h * 24 + slot, :] = cinv
                    return _
                lax.fori_loop(0, 8, prep_row, None)
                for d in range(3):
                    pltpu.async_copy(
                        h0_hbm.at[idxnf.at[h, pl.ds(d * 128, 128)]],
                        nfbuf.at[h, pl.ds(d * 128, 128)], semN
                    )

            def drain_half(h):
                for d in range(3):
                    pltpu.make_async_copy(
                        h0_hbm.at[idxnf.at[h, pl.ds(d * 128, 128)]],
                        nfbuf.at[h, pl.ds(d * 128, 128)], semN
                    ).wait()

            def unpack_row(ref, row):
                out = []
                for half2 in range(2):
                    v = ref[row, pl.ds(half2 * 16, 16)]
                    vb = plsc.bitcast(v, jnp.bfloat16)
                    a, b = plsc.unpack(vb, format=plsc.PackFormat.INTERLEAVED)
                    out.append(a)
                    out.append(b)
                return out

            def acc_half(h):
                def acc_row(j2, _):
                    sv = unpack_row(selfbuf.at[sl], h * 8 + j2)
                    i1 = [jnp.zeros((16,), jnp.float32) for _ in range(4)]
                    i2 = [jnp.zeros((16,), jnp.float32) for _ in range(4)]
                    for r in range(NREL):
                        slot = j2 * NREL + r
                        ci = cinvbuf[h * 24 + slot, :]
                        base = slot * 16
                        acc = [jnp.zeros((16,), jnp.float32) for _ in range(4)]
                        for k in range(16):
                            row = unpack_row(nfbuf.at[h], base + k)
                            for c in range(4):
                                acc[c] = acc[c] + row[c]
                        for c in range(4):
                            f1 = acc[c] * ci
                            wa = w1a_v[r, pl.ds(c * 16, 16)]
                            wb = w1b_v[r, pl.ds(c * 16, 16)]
                            i1[c] = i1[c] + f1 * wa
                            i2[c] = i2[c] + (sv[c] - f1) * wb
                    j = h * 8 + j2
                    for c in range(4):
                        outbuf[sl, j, pl.ds(c * 16, 16)] = sv[c]
                        outbuf[sl, j, pl.ds(ED + c * 16, 16)] = i1[c]
                        outbuf[sl, j, pl.ds(2 * ED + c * 16, 16)] = i2[c]
                    return _
                lax.fori_loop(0, 8, acc_row, None)

            fire_half(0)
            fire_half(1)
            ggn = gg + NT
            @pl.when(ggn < NGRP)
            def _():
                fire_group(ggn, (it + 1) % 2)
            pltpu.make_async_copy(h0_hbm.at[iota], selfbuf.at[sl], semS).wait()
            drain_half(0)
            acc_half(0)
            drain_half(1)
            acc_half(1)
            @pl.when(it >= 2)
            def _():
                pltpu.make_async_copy(
                    outbuf.at[sl], h1_hbm.at[pl.ds(0, 16)], semO
                ).wait()
            pltpu.async_copy(outbuf.at[sl], h1_hbm.at[pl.ds(gg * 16, 16)], semO)
        return _

    lax.fori_loop(0, GPT, group_body, None)

    for s in range(2):
        pltpu.make_async_copy(
            outbuf.at[s], h1_hbm.at[pl.ds(0, 16)], semO
        ).wait()

    @pl.when(wid == 0)
    def _():
        z = jnp.zeros((16,), jnp.float32)
        def zero_row(j, _):
            for c in range(12):
                outbuf[0, j, pl.ds(c * 16, 16)] = z
            return _
        lax.fori_loop(0, 8, zero_row, None)
        pltpu.sync_copy(outbuf.at[0, pl.ds(0, 8)], h1_hbm.at[pl.ds(SL, 8)])


def _run_k2(h0, nodes2, adj2d, w1a, w1b):
    mesh = plsc.VectorSubcoreMesh(core_axis_name="c", subcore_axis_name="s")
    f = pl.kernel(
        _k2_body,
        out_type=jax.ShapeDtypeStruct((SLP, 3 * ED), jnp.float32),
        mesh=mesh,
        compiler_params=pltpu.CompilerParams(needs_layout_passes=False, use_tc_tiling_on_sc=False),
        scratch_types=[
            pltpu.VMEM((NGRP, 16), jnp.int32),
            pltpu.VMEM((NREL * 256,), jnp.int32),
            pltpu.VMEM((NREL, ED), jnp.float32),
            pltpu.VMEM((NREL, ED), jnp.float32),
            pltpu.VMEM((2, NREL, 16, 16), jnp.int32),
            pltpu.VMEM((2, 16, ED // 2), jnp.int32),
            pltpu.VMEM((2, 384, ED // 2), jnp.int32),
            pltpu.VMEM((2, 384), jnp.int32),
            pltpu.VMEM((48, 16), jnp.float32),
            pltpu.VMEM((2, 16, 3 * ED), jnp.float32),
            pltpu.VMEM((N,), jnp.int32),
            pltpu.SemaphoreType.DMA,
            pltpu.SemaphoreType.DMA,
            pltpu.SemaphoreType.DMA,
            pltpu.SemaphoreType.DMA,
        ],
    )
    return f(h0, nodes2, adj2d, w1a, w1b)


def _k3_body(h1_hbm, nodes2_hbm, adj2d_hbm, w2a_hbm, w2b_hbm, x_hbm,
             nodes_v, w2a_v, w2b_v, adjbuf, selfbuf, nfbuf, cinvbuf,
             xbuf, T, semA, semN):
    wid = lax.axis_index("s") * 2 + lax.axis_index("c")
    iota = lax.iota(jnp.int32, 16)

    pltpu.sync_copy(nodes2_hbm, nodes_v)
    pltpu.sync_copy(w2a_hbm, w2a_v)
    pltpu.sync_copy(w2b_hbm, w2b_v)
    ids_vec = nodes_v[wid // 2, :]
    dA = [
        pltpu.async_copy(adj2d_hbm.at[ids_vec + r * N], adjbuf.at[r], semA)
        for r in range(NREL)
    ]
    pltpu.sync_copy(h1_hbm.at[pl.ds(wid * 8, 8)], selfbuf)
    for d in dA:
        d.wait()

    for half in range(2):
        def fire_node(l2, _):
            lane = (wid % 2) * 8 + half * 4 + l2
            i_node = wid * 8 + half * 4 + l2
            for r in range(NREL):
                neigh = adjbuf[r, lane, :]
                win, cinv = _dedup(neigh, iota, T)
                pos = B + r * B * K + i_node * K + iota
                idx2 = jnp.where(win, pos, jnp.full((16,), SL, jnp.int32))
                slot = l2 * NREL + r
                pltpu.async_copy(h1_hbm.at[idx2], nfbuf.at[slot], semN)
                cinvbuf[slot, :] = cinv
            return _

        lax.fori_loop(0, 4, fire_node, None)
        for s in range(12):
            pltpu.make_async_copy(h1_hbm.at[iota], nfbuf.at[s], semN).wait()

        def acc_node(l2, _):
            lrow = half * 4 + l2
            sf = [selfbuf[lrow, pl.ds(ED + c * 16, 16)] for c in range(8)]
            i1 = [jnp.zeros((16,), jnp.float32) for _ in range(8)]
            i2 = [jnp.zeros((16,), jnp.float32) for _ in range(8)]
            for r in range(NREL):
                slot = l2 * NREL + r
                ci = cinvbuf[slot, :]
                for c in range(8):
                    a = nfbuf[slot, 0, pl.ds(ED + c * 16, 16)]
                    for k in range(1, 16):
                        a = a + nfbuf[slot, k, pl.ds(ED + c * 16, 16)]
                    f1 = a * ci
                    wa = w2a_v[r, pl.ds(c * 16, 16)]
                    wb = w2b_v[r, pl.ds(c * 16, 16)]
                    i1[c] = i1[c] + f1 * wa
                    i2[c] = i2[c] + (sf[c] - f1) * wb
            for c in range(12):
                xbuf[lrow, pl.ds(c * 16, 16)] = selfbuf[lrow, pl.ds(c * 16, 16)]
            for c in range(8):
                xbuf[lrow, pl.ds(3 * ED + c * 16, 16)] = i1[c]
                xbuf[lrow, pl.ds(5 * ED + c * 16, 16)] = i2[c]
            return _

        lax.fori_loop(0, 4, acc_node, None)
    pltpu.sync_copy(xbuf, x_hbm.at[pl.ds(wid * 8, 8)])


def _run_k3(h1, nodes2, adj2d, w2a, w2b):
    mesh = plsc.VectorSubcoreMesh(core_axis_name="c", subcore_axis_name="s")
    f = pl.kernel(
        _k3_body,
        out_type=jax.ShapeDtypeStruct((B, 7 * ED), jnp.float32),
        mesh=mesh,
        compiler_params=pltpu.CompilerParams(needs_layout_passes=False, use_tc_tiling_on_sc=False),
        scratch_types=[
            pltpu.VMEM((16, 16), jnp.int32),
            pltpu.VMEM((NREL, 2 * ED), jnp.float32),
            pltpu.VMEM((NREL, 2 * ED), jnp.float32),
            pltpu.VMEM((NREL, 16, 16), jnp.int32),
            pltpu.VMEM((8, 3 * ED), jnp.float32),
            pltpu.VMEM((12, 16, 3 * ED), jnp.float32),
            pltpu.VMEM((12, 16), jnp.float32),
            pltpu.VMEM((8, 7 * ED), jnp.float32),
            pltpu.VMEM((N,), jnp.int32),
            pltpu.SemaphoreType.DMA,
            pltpu.SemaphoreType.DMA,
        ],
    )
    return f(h1, nodes2, adj2d, w2a, w2b)


def _k4_body(x_ref, w_ref, pg_ref, b_ref, out_ref):
    s = jnp.dot(x_ref[...], w_ref[...], preferred_element_type=jnp.float32)
    out_ref[...] = s + jnp.log(pg_ref[...]) + b_ref[...][0:1, :]


def _run_k4(x, W_out_pad, pg_pad, bq):
    return pl.pallas_call(
        _k4_body,
        out_shape=jax.ShapeDtypeStruct((B, 128), jnp.float32),
    )(x, W_out_pad, pg_pad, bq)


def kernel(nodes, feat_data, adj_lists, prior, W0, b0, alpha1, alpha2,
           W_out, b_out):
    nodes2 = nodes.reshape(16, 16)
    adj2d = adj_lists.reshape(NREL * N, K)

    a1t8 = jnp.zeros((8, 2 * ED * 2), jnp.float32)
    a1t8 = a1t8.at[0:NREL, 0:2 * ED].set(alpha1.T)
    a2t8 = jnp.zeros((8, 2 * ED * 2), jnp.float32)
    a2t8 = a2t8.at[4:4 + NREL, :].set(alpha2.T)

    perm = jnp.array(PERM64, jnp.int32)
    h0w, wcat = _run_k1(feat_data, W0[:, perm], b0[perm], a1t8, a2t8)
    h0 = lax.bitcast_convert_type(
        h0w.reshape(NP, ED // 2, 2), jnp.int32
    )
    w1a = wcat[0:NREL, 0:ED]
    w1b = wcat[0:NREL, ED:2 * ED]
    w2a = wcat[4:4 + NREL, 0:2 * ED]
    w2b = wcat[4:4 + NREL, 2 * ED:]

    h1 = _run_k2(h0, nodes2, adj2d, w1a, w1b)
    x = _run_k3(h1, nodes2, adj2d, w2a, w2b)

    pg_pad = jnp.concatenate(
        [prior[nodes], jnp.ones((B, 126), jnp.float32)], axis=1
    )
    bq = jnp.broadcast_to(jnp.pad(b_out, (0, 126))[None, :], (8, 128))
    out = _run_k4(x, jnp.pad(W_out, ((0, 0), (0, 126))), pg_pad, bq)
    return out[:, :2]


# speedup vs baseline: 7.5002x; 7.5002x over previous
import functools

import jax
import jax.numpy as jnp
from jax import lax
from jax.experimental import pallas as pl
from jax.experimental.pallas import tpu as pltpu
from jax.experimental.pallas import tpu_sc as plsc

N = 50000
D = 128
K = 16
ED = 64
B = 256
NREL = 3
SL = B + NREL * B * K
SLP = SL + 8
NT = 32
NGRP = SL // K
GPT = (NGRP + NT - 1) // NT

BLK1 = 512
NPH = 49 * BLK1
NP = 2 * NPH
ZROW = 2 * (N - NPH) + 1

PERM64 = [32 * b + (i // 2 if i % 2 == 0 else 16 + i // 2)
          for b in (0, 1) for i in range(32)]


def _k1_body(fa_ref, fb_ref, w0_ref, b0_ref, a1_ref, a2_ref, h0_ref, w_ref):
    pid = pl.program_id(0)
    ha = jnp.dot(fa_ref[...], w0_ref[...], preferred_element_type=jnp.float32)
    hb = jnp.dot(fb_ref[...], w0_ref[...], preferred_element_type=jnp.float32)
    bias = b0_ref[...][0:1, :]
    rowb = NPH + pid * BLK1 + lax.broadcasted_iota(jnp.int32, (BLK1, ED), 0)
    hb = jnp.where(rowb < N, hb + bias, 0.0)
    h0_ref[...] = jnp.concatenate([ha + bias, hb], axis=1).astype(jnp.bfloat16)

    @pl.when(pid == 0)
    def _():
        rid1 = lax.broadcasted_iota(jnp.int32, (8, 2 * ED * 2), 0)
        a1 = a1_ref[...]
        v1 = rid1 < NREL
        m1 = jnp.max(jnp.where(v1, a1, -jnp.inf), axis=0, keepdims=True)
        e1 = jnp.where(v1, jnp.exp(a1 - m1), 0.0)
        w1 = e1 / jnp.sum(e1, axis=0, keepdims=True)
        a2 = a2_ref[...]
        v2 = (rid1 >= 4) & (rid1 < 4 + NREL)
        m2 = jnp.max(jnp.where(v2, a2, -jnp.inf), axis=0, keepdims=True)
        e2 = jnp.where(v2, jnp.exp(a2 - m2), 0.0)
        w2 = e2 / jnp.sum(e2, axis=0, keepdims=True)
        w_ref[...] = jnp.where(v1, w1, 0.0) + jnp.where(v2, w2, 0.0)


def _run_k1(feat_data, W0, b0, a1t8, a2t8):
    b0b = jnp.broadcast_to(b0[None, :], (8, ED))
    nb = NPH // BLK1
    return pl.pallas_call(
        _k1_body,
        grid=(nb,),
        in_specs=[
            pl.BlockSpec((BLK1, D), lambda i: (i, 0)),
            pl.BlockSpec((BLK1, D), lambda i: (i + NPH // BLK1, 0)),
            pl.BlockSpec((D, ED), lambda i: (0, 0)),
            pl.BlockSpec((8, ED), lambda i: (0, 0)),
            pl.BlockSpec((8, 2 * ED * 2), lambda i: (0, 0)),
            pl.BlockSpec((8, 2 * ED * 2), lambda i: (0, 0)),
        ],
        out_specs=[
            pl.BlockSpec((BLK1, 2 * ED), lambda i: (i, 0)),
            pl.BlockSpec((8, 2 * ED * 2), lambda i: (0, 0)),
        ],
        out_shape=[
            jax.ShapeDtypeStruct((NPH, 2 * ED), jnp.bfloat16),
            jax.ShapeDtypeStruct((8, 2 * ED * 2), jnp.float32),
        ],
    )(feat_data, feat_data, W0, b0b, a1t8, a2t8)


def _dedup(neigh, iota, T):
    plsc.store_scatter(T, [neigh], iota)
    g = plsc.load_gather(T, [neigh])
    win = g == iota
    cnt = plsc.all_reduce_population_count(win)
    cinv = 1.0 / cnt.astype(jnp.float32)
    return win, cinv


def _k2_body(h0_hbm, nodes2_hbm, adj2d_hbm, w1a_hbm, w1b_hbm, h1_hbm,
             sids2, idxbuf, w1a_v, w1b_v, adjbuf, selfbuf, nfbuf, idxnf,
             cinvbuf, outbuf, T, semA, semS, semN, semO):
    wid = lax.axis_index("s") * 2 + lax.axis_index("c")
    iota = lax.iota(jnp.int32, 16)

    pltpu.sync_copy(nodes2_hbm, sids2.at[pl.ds(0, 16)])
    dStage = []
    for r in range(NREL):
        def _mk_idx(q, _):
            idxbuf[pl.ds(r * 256 + q * 16, 16)] = sids2[q, :] + r * N
            return _
        lax.fori_loop(0, 16, _mk_idx, None)
        dStage.append(pltpu.async_copy(
            adj2d_hbm.at[idxbuf.at[pl.ds(r * 256, 256)]],
            sids2.at[pl.ds(16 + r * 256, 256)], semA
        ))
    pltpu.sync_copy(w1a_hbm, w1a_v)
    pltpu.sync_copy(w1b_hbm, w1b_v)
    for d in dStage:
        d.wait()

    def pairmap(n):
        return jnp.where(n < NPH, 2 * n, 2 * n - (2 * NPH - 1))

    def fire_group(gg, sl):
        ids_vec = sids2[gg, :]
        pltpu.async_copy(h0_hbm.at[pairmap(ids_vec)], selfbuf.at[sl], semS)
        for r in range(NREL):
            pltpu.async_copy(
                adj2d_hbm.at[ids_vec + r * N], adjbuf.at[sl, r], semA
            )

    @pl.when(wid < NGRP)
    def _():
        fire_group(wid, 0)

    def group_body(it, _):
        gg = wid + it * NT
        sl = it % 2

        @pl.when(gg < NGRP)
        def _():
            for r in range(NREL):
                pltpu.make_async_copy(
                    adj2d_hbm.at[iota], adjbuf.at[sl, r], semA
                ).wait()

            def fire_half(h):
                def prep_row(j2, _):
                    for r in range(NREL):
                        neigh = adjbuf[sl, r, h * 8 + j2, :]
                        win, cinv = _dedup(neigh, iota, T)
                        idx2 = jnp.where(win, pairmap(neigh),
                                         jnp.full((16,), ZROW, jnp.int32))
                        slot = j2 * NREL + r
                        idxnf[h, pl.ds(slot * 16, 16)] = idx2
                        cinvbuf[h * 24 + slot, :] = cinv
                    return _
                lax.fori_loop(0, 8, prep_row, None)
                for d in range(3):
                    pltpu.async_copy(
                        h0_hbm.at[idxnf.at[h, pl.ds(d * 128, 128)]],
                        nfbuf.at[h, pl.ds(d * 128, 128)], semN
                    )

            def drain_half(h):
                for d in range(3):
                    pltpu.make_async_copy(
                        h0_hbm.at[idxnf.at[h, pl.ds(d * 128, 128)]],
                        nfbuf.at[h, pl.ds(d * 128, 128)], semN
                    ).wait()

            def unpack_row(ref, row):
                out = []
                for half2 in range(2):
                    v = ref[row, pl.ds(half2 * 32, 32)]
                    a, b = plsc.unpack(v, format=plsc.PackFormat.INTERLEAVED)
                    out.append(a)
                    out.append(b)
                return out

            def acc_half(h):
                def acc_row(j2, _):
                    sv = unpack_row(selfbuf.at[sl], h * 8 + j2)
                    i1 = [jnp.zeros((16,), jnp.float32) for _ in range(4)]
                    i2 = [jnp.zeros((16,), jnp.float32) for _ in range(4)]
                    for r in range(NREL):
                        slot = j2 * NREL + r
                        ci = cinvbuf[h * 24 + slot, :]
                        base = slot * 16
                        acc = [jnp.zeros((16,), jnp.float32) for _ in range(4)]
                        for k in range(16):
                            row = unpack_row(nfbuf.at[h], base + k)
                            for c in range(4):
                                acc[c] = acc[c] + row[c]
                        for c in range(4):
                            f1 = acc[c] * ci
                            wa = w1a_v[r, pl.ds(c * 16, 16)]
                            wb = w1b_v[r, pl.ds(c * 16, 16)]
                            i1[c] = i1[c] + f1 * wa
                            i2[c] = i2[c] + (sv[c] - f1) * wb
                    j = h * 8 + j2
                    for c in range(4):
                        outbuf[sl, j, pl.ds(c * 16, 16)] = sv[c]
                        outbuf[sl, j, pl.ds(ED + c * 16, 16)] = i1[c]
                        outbuf[sl, j, pl.ds(2 * ED + c * 16, 16)] = i2[c]
                    return _
                lax.fori_loop(0, 8, acc_row, None)

            fire_half(0)
            fire_half(1)
            ggn = gg + NT
            @pl.when(ggn < NGRP)
            def _():
                fire_group(ggn, (it + 1) % 2)
            pltpu.make_async_copy(h0_hbm.at[iota], selfbuf.at[sl], semS).wait()
            drain_half(0)
            acc_half(0)
            drain_half(1)
            acc_half(1)
            @pl.when(it >= 2)
            def _():
                pltpu.make_async_copy(
                    outbuf.at[sl], h1_hbm.at[pl.ds(0, 16)], semO
                ).wait()
            pltpu.async_copy(outbuf.at[sl], h1_hbm.at[pl.ds(gg * 16, 16)], semO)
        return _

    lax.fori_loop(0, GPT, group_body, None)

    for s in range(2):
        pltpu.make_async_copy(
            outbuf.at[s], h1_hbm.at[pl.ds(0, 16)], semO
        ).wait()

    @pl.when(wid == 0)
    def _():
        z = jnp.zeros((16,), jnp.float32)
        def zero_row(j, _):
            for c in range(12):
                outbuf[0, j, pl.ds(c * 16, 16)] = z
            return _
        lax.fori_loop(0, 8, zero_row, None)
        pltpu.sync_copy(outbuf.at[0, pl.ds(0, 8)], h1_hbm.at[pl.ds(SL, 8)])


def _run_k2(h0, nodes2, adj2d, w1a, w1b):
    mesh = plsc.VectorSubcoreMesh(core_axis_name="c", subcore_axis_name="s")
    f = pl.kernel(
        _k2_body,
        out_type=jax.ShapeDtypeStruct((SLP, 3 * ED), jnp.float32),
        mesh=mesh,
        compiler_params=pltpu.CompilerParams(needs_layout_passes=False, use_tc_tiling_on_sc=False),
        scratch_types=[
            pltpu.VMEM((NGRP, 16), jnp.int32),
            pltpu.VMEM((NREL * 256,), jnp.int32),
            pltpu.VMEM((NREL, ED), jnp.float32),
            pltpu.VMEM((NREL, ED), jnp.float32),
            pltpu.VMEM((2, NREL, 16, 16), jnp.int32),
            pltpu.VMEM((2, 16, ED), jnp.bfloat16),
            pltpu.VMEM((2, 384, ED), jnp.bfloat16),
            pltpu.VMEM((2, 384), jnp.int32),
            pltpu.VMEM((48, 16), jnp.float32),
            pltpu.VMEM((2, 16, 3 * ED), jnp.float32),
            pltpu.VMEM((N,), jnp.int32),
            pltpu.SemaphoreType.DMA,
            pltpu.SemaphoreType.DMA,
            pltpu.SemaphoreType.DMA,
            pltpu.SemaphoreType.DMA,
        ],
    )
    return f(h0, nodes2, adj2d, w1a, w1b)


def _k3_body(h1_hbm, nodes2_hbm, adj2d_hbm, w2a_hbm, w2b_hbm, x_hbm,
             nodes_v, w2a_v, w2b_v, adjbuf, selfbuf, nfbuf, cinvbuf,
             xbuf, T, semA, semN):
    wid = lax.axis_index("s") * 2 + lax.axis_index("c")
    iota = lax.iota(jnp.int32, 16)

    pltpu.sync_copy(nodes2_hbm, nodes_v)
    pltpu.sync_copy(w2a_hbm, w2a_v)
    pltpu.sync_copy(w2b_hbm, w2b_v)
    ids_vec = nodes_v[wid // 2, :]
    dA = [
        pltpu.async_copy(adj2d_hbm.at[ids_vec + r * N], adjbuf.at[r], semA)
        for r in range(NREL)
    ]
    pltpu.sync_copy(h1_hbm.at[pl.ds(wid * 8, 8)], selfbuf)
    for d in dA:
        d.wait()

    for half in range(2):
        def fire_node(l2, _):
            lane = (wid % 2) * 8 + half * 4 + l2
            i_node = wid * 8 + half * 4 + l2
            for r in range(NREL):
                neigh = adjbuf[r, lane, :]
                win, cinv = _dedup(neigh, iota, T)
                pos = B + r * B * K + i_node * K + iota
                idx2 = jnp.where(win, pos, jnp.full((16,), SL, jnp.int32))
                slot = l2 * NREL + r
                pltpu.async_copy(h1_hbm.at[idx2], nfbuf.at[slot], semN)
                cinvbuf[slot, :] = cinv
            return _

        lax.fori_loop(0, 4, fire_node, None)
        for s in range(12):
            pltpu.make_async_copy(h1_hbm.at[iota], nfbuf.at[s], semN).wait()

        def acc_node(l2, _):
            lrow = half * 4 + l2
            sf = [selfbuf[lrow, pl.ds(ED + c * 16, 16)] for c in range(8)]
            i1 = [jnp.zeros((16,), jnp.float32) for _ in range(8)]
            i2 = [jnp.zeros((16,), jnp.float32) for _ in range(8)]
            for r in range(NREL):
                slot = l2 * NREL + r
                ci = cinvbuf[slot, :]
                for c in range(8):
                    a = nfbuf[slot, 0, pl.ds(ED + c * 16, 16)]
                    for k in range(1, 16):
                        a = a + nfbuf[slot, k, pl.ds(ED + c * 16, 16)]
                    f1 = a * ci
                    wa = w2a_v[r, pl.ds(c * 16, 16)]
                    wb = w2b_v[r, pl.ds(c * 16, 16)]
                    i1[c] = i1[c] + f1 * wa
                    i2[c] = i2[c] + (sf[c] - f1) * wb
            for c in range(12):
                xbuf[lrow, pl.ds(c * 16, 16)] = selfbuf[lrow, pl.ds(c * 16, 16)]
            for c in range(8):
                xbuf[lrow, pl.ds(3 * ED + c * 16, 16)] = i1[c]
                xbuf[lrow, pl.ds(5 * ED + c * 16, 16)] = i2[c]
            return _

        lax.fori_loop(0, 4, acc_node, None)
    pltpu.sync_copy(xbuf, x_hbm.at[pl.ds(wid * 8, 8)])


def _run_k3(h1, nodes2, adj2d, w2a, w2b):
    mesh = plsc.VectorSubcoreMesh(core_axis_name="c", subcore_axis_name="s")
    f = pl.kernel(
        _k3_body,
        out_type=jax.ShapeDtypeStruct((B, 7 * ED), jnp.float32),
        mesh=mesh,
        compiler_params=pltpu.CompilerParams(needs_layout_passes=False, use_tc_tiling_on_sc=False),
        scratch_types=[
            pltpu.VMEM((16, 16), jnp.int32),
            pltpu.VMEM((NREL, 2 * ED), jnp.float32),
            pltpu.VMEM((NREL, 2 * ED), jnp.float32),
            pltpu.VMEM((NREL, 16, 16), jnp.int32),
            pltpu.VMEM((8, 3 * ED), jnp.float32),
            pltpu.VMEM((12, 16, 3 * ED), jnp.float32),
            pltpu.VMEM((12, 16), jnp.float32),
            pltpu.VMEM((8, 7 * ED), jnp.float32),
            pltpu.VMEM((N,), jnp.int32),
            pltpu.SemaphoreType.DMA,
            pltpu.SemaphoreType.DMA,
        ],
    )
    return f(h1, nodes2, adj2d, w2a, w2b)


def _k4_body(x_ref, w_ref, pg_ref, b_ref, out_ref):
    s = jnp.dot(x_ref[...], w_ref[...], preferred_element_type=jnp.float32)
    out_ref[...] = s + jnp.log(pg_ref[...]) + b_ref[...][0:1, :]


def _run_k4(x, W_out_pad, pg_pad, bq):
    return pl.pallas_call(
        _k4_body,
        out_shape=jax.ShapeDtypeStruct((B, 128), jnp.float32),
    )(x, W_out_pad, pg_pad, bq)


def kernel(nodes, feat_data, adj_lists, prior, W0, b0, alpha1, alpha2,
           W_out, b_out):
    nodes2 = nodes.reshape(16, 16)
    adj2d = adj_lists.reshape(NREL * N, K)

    a1t8 = jnp.zeros((8, 2 * ED * 2), jnp.float32)
    a1t8 = a1t8.at[0:NREL, 0:2 * ED].set(alpha1.T)
    a2t8 = jnp.zeros((8, 2 * ED * 2), jnp.float32)
    a2t8 = a2t8.at[4:4 + NREL, :].set(alpha2.T)

    perm = jnp.array(PERM64, jnp.int32)
    h0w, wcat = _run_k1(feat_data, W0[:, perm], b0[perm], a1t8, a2t8)
    h0 = h0w.reshape(NP, ED)
    w1a = wcat[0:NREL, 0:ED]
    w1b = wcat[0:NREL, ED:2 * ED]
    w2a = wcat[4:4 + NREL, 0:2 * ED]
    w2b = wcat[4:4 + NREL, 2 * ED:]

    h1 = _run_k2(h0, nodes2, adj2d, w1a, w1b)
    x = _run_k3(h1, nodes2, adj2d, w2a, w2b)

    pg_pad = jnp.concatenate(
        [prior[nodes], jnp.ones((B, 126), jnp.float32)], axis=1
    )
    bq = jnp.broadcast_to(jnp.pad(b_out, (0, 126))[None, :], (8, 128))
    out = _run_k4(x, jnp.pad(W_out, ((0, 0), (0, 126))), pg_pad, bq)
    return out[:, :2]


# speedup vs baseline: 8.1558x; 1.0874x over previous
import functools

import jax
import jax.numpy as jnp
from jax import lax
from jax.experimental import pallas as pl
from jax.experimental.pallas import tpu as pltpu
from jax.experimental.pallas import tpu_sc as plsc

N = 50000
D = 128
K = 16
ED = 64
B = 256
NREL = 3
SL = B + NREL * B * K
SLP = SL + 8
NT = 32
NGRP = SL // K
GPT = (NGRP + NT - 1) // NT

BLK1 = 512
NPH = 49 * BLK1
NP = 2 * NPH
ZROW = 2 * (N - NPH) + 1


def _k1_body(fa_ref, fb_ref, w0_ref, b0_ref, a1_ref, a2_ref, h0_ref, w_ref):
    pid = pl.program_id(0)
    ha = jnp.dot(fa_ref[...], w0_ref[...], preferred_element_type=jnp.float32)
    hb = jnp.dot(fb_ref[...], w0_ref[...], preferred_element_type=jnp.float32)
    bias = b0_ref[...][0:1, :]
    rowb = NPH + pid * BLK1 + lax.broadcasted_iota(jnp.int32, (BLK1, ED), 0)
    hb = jnp.where(rowb < N, hb + bias, 0.0)
    h0_ref[...] = jnp.concatenate([ha + bias, hb], axis=1)

    @pl.when(pid == 0)
    def _():
        rid1 = lax.broadcasted_iota(jnp.int32, (8, 2 * ED * 2), 0)
        a1 = a1_ref[...]
        v1 = rid1 < NREL
        m1 = jnp.max(jnp.where(v1, a1, -jnp.inf), axis=0, keepdims=True)
        e1 = jnp.where(v1, jnp.exp(a1 - m1), 0.0)
        w1 = e1 / jnp.sum(e1, axis=0, keepdims=True)
        a2 = a2_ref[...]
        v2 = (rid1 >= 4) & (rid1 < 4 + NREL)
        m2 = jnp.max(jnp.where(v2, a2, -jnp.inf), axis=0, keepdims=True)
        e2 = jnp.where(v2, jnp.exp(a2 - m2), 0.0)
        w2 = e2 / jnp.sum(e2, axis=0, keepdims=True)
        w_ref[...] = jnp.where(v1, w1, 0.0) + jnp.where(v2, w2, 0.0)


def _run_k1(feat_data, W0, b0, a1t8, a2t8):
    b0b = jnp.broadcast_to(b0[None, :], (8, ED))
    nb = NPH // BLK1
    return pl.pallas_call(
        _k1_body,
        grid=(nb,),
        in_specs=[
            pl.BlockSpec((BLK1, D), lambda i: (i, 0)),
            pl.BlockSpec((BLK1, D), lambda i: (i + NPH // BLK1, 0)),
            pl.BlockSpec((D, ED), lambda i: (0, 0)),
            pl.BlockSpec((8, ED), lambda i: (0, 0)),
            pl.BlockSpec((8, 2 * ED * 2), lambda i: (0, 0)),
            pl.BlockSpec((8, 2 * ED * 2), lambda i: (0, 0)),
        ],
        out_specs=[
            pl.BlockSpec((BLK1, 2 * ED), lambda i: (i, 0)),
            pl.BlockSpec((8, 2 * ED * 2), lambda i: (0, 0)),
        ],
        out_shape=[
            jax.ShapeDtypeStruct((NPH, 2 * ED), jnp.float32),
            jax.ShapeDtypeStruct((8, 2 * ED * 2), jnp.float32),
        ],
    )(feat_data, feat_data, W0, b0b, a1t8, a2t8)


def _k15_body(h0w_hbm, tab_hbm, inbuf, outbuf, unused_sem):
    wid = lax.axis_index("s") * 2 + lax.axis_index("c")
    CH = 196
    for chunk in range(4):
        a = wid * 784 + chunk * CH
        pltpu.sync_copy(h0w_hbm.at[pl.ds(a, CH)], inbuf)

        def row_body(j, _):
            for half in range(2):
                for q in range(2):
                    c0 = inbuf[j, pl.ds(half * 64 + q * 32, 16)]
                    c1 = inbuf[j, pl.ds(half * 64 + q * 32 + 16, 16)]
                    p = plsc.pack(c0, c1, format=plsc.PackFormat.INTERLEAVED)
                    outbuf[2 * j + half, pl.ds(q * 32, 32)] = p
            return _

        lax.fori_loop(0, CH, row_body, None)
        pltpu.sync_copy(outbuf, tab_hbm.at[pl.ds(2 * a, 2 * CH)])


def _run_k15(h0w):
    mesh = plsc.VectorSubcoreMesh(core_axis_name="c", subcore_axis_name="s")
    f = pl.kernel(
        _k15_body,
        out_type=jax.ShapeDtypeStruct((NP, ED), jnp.bfloat16),
        mesh=mesh,
        compiler_params=pltpu.CompilerParams(
            needs_layout_passes=False, use_tc_tiling_on_sc=False),
        scratch_types=[
            pltpu.VMEM((196, 2 * ED), jnp.float32),
            pltpu.VMEM((392, ED), jnp.bfloat16),
            pltpu.SemaphoreType.DMA,
        ],
    )
    return f(h0w)


def _dedup(neigh, iota, T):
    plsc.store_scatter(T, [neigh], iota)
    g = plsc.load_gather(T, [neigh])
    win = g == iota
    cnt = plsc.all_reduce_population_count(win)
    cinv = 1.0 / cnt.astype(jnp.float32)
    return win, cinv


def _k2_body(h0_hbm, nodes2_hbm, adj2d_hbm, w1a_hbm, w1b_hbm, h1_hbm,
             sids2, idxbuf, w1a_v, w1b_v, adjbuf, selfbuf, nfbuf, idxnf,
             cinvbuf, outbuf, T, semA, semS, semN, semO):
    wid = lax.axis_index("s") * 2 + lax.axis_index("c")
    iota = lax.iota(jnp.int32, 16)

    pltpu.sync_copy(nodes2_hbm, sids2.at[pl.ds(0, 16)])
    dStage = []
    for r in range(NREL):
        def _mk_idx(q, _):
            idxbuf[pl.ds(r * 256 + q * 16, 16)] = sids2[q, :] + r * N
            return _
        lax.fori_loop(0, 16, _mk_idx, None)
        dStage.append(pltpu.async_copy(
            adj2d_hbm.at[idxbuf.at[pl.ds(r * 256, 256)]],
            sids2.at[pl.ds(16 + r * 256, 256)], semA
        ))
    pltpu.sync_copy(w1a_hbm, w1a_v)
    pltpu.sync_copy(w1b_hbm, w1b_v)
    for d in dStage:
        d.wait()

    def pairmap(n):
        return jnp.where(n < NPH, 2 * n, 2 * n - (2 * NPH - 1))

    def fire_group(gg, sl):
        ids_vec = sids2[gg, :]
        pltpu.async_copy(h0_hbm.at[pairmap(ids_vec)], selfbuf.at[sl], semS)
        for r in range(NREL):
            pltpu.async_copy(
                adj2d_hbm.at[ids_vec + r * N], adjbuf.at[sl, r], semA
            )

    @pl.when(wid < NGRP)
    def _():
        fire_group(wid, 0)

    def group_body(it, _):
        gg = wid + it * NT
        sl = it % 2

        @pl.when(gg < NGRP)
        def _():
            for r in range(NREL):
                pltpu.make_async_copy(
                    adj2d_hbm.at[iota], adjbuf.at[sl, r], semA
                ).wait()

            def fire_half(h):
                def prep_row(j2, _):
                    for r in range(NREL):
                        neigh = adjbuf[sl, r, h * 8 + j2, :]
                        win, cinv = _dedup(neigh, iota, T)
                        idx2 = jnp.where(win, pairmap(neigh),
                                         jnp.full((16,), ZROW, jnp.int32))
                        slot = j2 * NREL + r
                        idxnf[h, pl.ds(slot * 16, 16)] = idx2
                        cinvbuf[h * 24 + slot, :] = cinv
                    return _
                lax.fori_loop(0, 8, prep_row, None)
                for d in range(3):
                    pltpu.async_copy(
                        h0_hbm.at[idxnf.at[h, pl.ds(d * 128, 128)]],
                        nfbuf.at[h, pl.ds(d * 128, 128)], semN
                    )

            def drain_half(h):
                for d in range(3):
                    pltpu.make_async_copy(
                        h0_hbm.at[idxnf.at[h, pl.ds(d * 128, 128)]],
                        nfbuf.at[h, pl.ds(d * 128, 128)], semN
                    ).wait()

            def unpack_row(ref, row):
                out = []
                for half2 in range(2):
                    v = ref[row, pl.ds(half2 * 32, 32)]
                    a, b = plsc.unpack(v, format=plsc.PackFormat.INTERLEAVED)
                    out.append(a)
                    out.append(b)
                return out

            def acc_half(h):
                def acc_row(j2, _):
                    sv = unpack_row(selfbuf.at[sl], h * 8 + j2)
                    i1 = [jnp.zeros((16,), jnp.float32) for _ in range(4)]
                    i2 = [jnp.zeros((16,), jnp.float32) for _ in range(4)]
                    for r in range(NREL):
                        slot = j2 * NREL + r
                        ci = cinvbuf[h * 24 + slot, :]
                        base = slot * 16
                        acc = [jnp.zeros((16,), jnp.float32) for _ in range(4)]
                        for k in range(16):
                            row = unpack_row(nfbuf.at[h], base + k)
                            for c in range(4):
                                acc[c] = acc[c] + row[c]
                        for c in range(4):
                            f1 = acc[c] * ci
                            wa = w1a_v[r, pl.ds(c * 16, 16)]
                            wb = w1b_v[r, pl.ds(c * 16, 16)]
                            i1[c] = i1[c] + f1 * wa
                            i2[c] = i2[c] + (sv[c] - f1) * wb
                    j = h * 8 + j2
                    for c in range(4):
                        outbuf[sl, j, pl.ds(c * 16, 16)] = sv[c]
                        outbuf[sl, j, pl.ds(ED + c * 16, 16)] = i1[c]
                        outbuf[sl, j, pl.ds(2 * ED + c * 16, 16)] = i2[c]
                    return _
                lax.fori_loop(0, 8, acc_row, None)

            fire_half(0)
            fire_half(1)
            ggn = gg + NT
            @pl.when(ggn < NGRP)
            def _():
                fire_group(ggn, (it + 1) % 2)
            pltpu.make_async_copy(h0_hbm.at[iota], selfbuf.at[sl], semS).wait()
            drain_half(0)
            acc_half(0)
            drain_half(1)
            acc_half(1)
            @pl.when(it >= 2)
            def _():
                pltpu.make_async_copy(
                    outbuf.at[sl], h1_hbm.at[pl.ds(0, 16)], semO
                ).wait()
            pltpu.async_copy(outbuf.at[sl], h1_hbm.at[pl.ds(gg * 16, 16)], semO)
        return _

    lax.fori_loop(0, GPT, group_body, None)

    for s in range(2):
        pltpu.make_async_copy(
            outbuf.at[s], h1_hbm.at[pl.ds(0, 16)], semO
        ).wait()

    @pl.when(wid == 0)
    def _():
        z = jnp.zeros((16,), jnp.float32)
        def zero_row(j, _):
            for c in range(12):
                outbuf[0, j, pl.ds(c * 16, 16)] = z
            return _
        lax.fori_loop(0, 8, zero_row, None)
        pltpu.sync_copy(outbuf.at[0, pl.ds(0, 8)], h1_hbm.at[pl.ds(SL, 8)])


def _run_k2(h0, nodes2, adj2d, w1a, w1b):
    mesh = plsc.VectorSubcoreMesh(core_axis_name="c", subcore_axis_name="s")
    f = pl.kernel(
        _k2_body,
        out_type=jax.ShapeDtypeStruct((SLP, 3 * ED), jnp.float32),
        mesh=mesh,
        compiler_params=pltpu.CompilerParams(needs_layout_passes=False, use_tc_tiling_on_sc=False),
        scratch_types=[
            pltpu.VMEM((NGRP, 16), jnp.int32),
            pltpu.VMEM((NREL * 256,), jnp.int32),
            pltpu.VMEM((NREL, ED), jnp.float32),
            pltpu.VMEM((NREL, ED), jnp.float32),
            pltpu.VMEM((2, NREL, 16, 16), jnp.int32),
            pltpu.VMEM((2, 16, ED), jnp.bfloat16),
            pltpu.VMEM((2, 384, ED), jnp.bfloat16),
            pltpu.VMEM((2, 384), jnp.int32),
            pltpu.VMEM((48, 16), jnp.float32),
            pltpu.VMEM((2, 16, 3 * ED), jnp.float32),
            pltpu.VMEM((N,), jnp.int32),
            pltpu.SemaphoreType.DMA,
            pltpu.SemaphoreType.DMA,
            pltpu.SemaphoreType.DMA,
            pltpu.SemaphoreType.DMA,
        ],
    )
    return f(h0, nodes2, adj2d, w1a, w1b)


def _k3_body(h1_hbm, nodes2_hbm, adj2d_hbm, w2a_hbm, w2b_hbm, x_hbm,
             nodes_v, w2a_v, w2b_v, adjbuf, selfbuf, nfbuf, cinvbuf,
             xbuf, T, semA, semN):
    wid = lax.axis_index("s") * 2 + lax.axis_index("c")
    iota = lax.iota(jnp.int32, 16)

    pltpu.sync_copy(nodes2_hbm, nodes_v)
    pltpu.sync_copy(w2a_hbm, w2a_v)
    pltpu.sync_copy(w2b_hbm, w2b_v)
    ids_vec = nodes_v[wid // 2, :]
    dA = [
        pltpu.async_copy(adj2d_hbm.at[ids_vec + r * N], adjbuf.at[r], semA)
        for r in range(NREL)
    ]
    pltpu.sync_copy(h1_hbm.at[pl.ds(wid * 8, 8)], selfbuf)
    for d in dA:
        d.wait()

    for half in range(2):
        def fire_node(l2, _):
            lane = (wid % 2) * 8 + half * 4 + l2
            i_node = wid * 8 + half * 4 + l2
            for r in range(NREL):
                neigh = adjbuf[r, lane, :]
                win, cinv = _dedup(neigh, iota, T)
                pos = B + r * B * K + i_node * K + iota
                idx2 = jnp.where(win, pos, jnp.full((16,), SL, jnp.int32))
                slot = l2 * NREL + r
                pltpu.async_copy(h1_hbm.at[idx2], nfbuf.at[slot], semN)
                cinvbuf[slot, :] = cinv
            return _

        lax.fori_loop(0, 4, fire_node, None)
        for s in range(12):
            pltpu.make_async_copy(h1_hbm.at[iota], nfbuf.at[s], semN).wait()

        def acc_node(l2, _):
            lrow = half * 4 + l2
            sf = [selfbuf[lrow, pl.ds(ED + c * 16, 16)] for c in range(8)]
            i1 = [jnp.zeros((16,), jnp.float32) for _ in range(8)]
            i2 = [jnp.zeros((16,), jnp.float32) for _ in range(8)]
            for r in range(NREL):
                slot = l2 * NREL + r
                ci = cinvbuf[slot, :]
                for c in range(8):
                    a = nfbuf[slot, 0, pl.ds(ED + c * 16, 16)]
                    for k in range(1, 16):
                        a = a + nfbuf[slot, k, pl.ds(ED + c * 16, 16)]
                    f1 = a * ci
                    wa = w2a_v[r, pl.ds(c * 16, 16)]
                    wb = w2b_v[r, pl.ds(c * 16, 16)]
                    i1[c] = i1[c] + f1 * wa
                    i2[c] = i2[c] + (sf[c] - f1) * wb
            for c in range(12):
                xbuf[lrow, pl.ds(c * 16, 16)] = selfbuf[lrow, pl.ds(c * 16, 16)]
            for c in range(8):
                xbuf[lrow, pl.ds(3 * ED + c * 16, 16)] = i1[c]
                xbuf[lrow, pl.ds(5 * ED + c * 16, 16)] = i2[c]
            return _

        lax.fori_loop(0, 4, acc_node, None)
    pltpu.sync_copy(xbuf, x_hbm.at[pl.ds(wid * 8, 8)])


def _run_k3(h1, nodes2, adj2d, w2a, w2b):
    mesh = plsc.VectorSubcoreMesh(core_axis_name="c", subcore_axis_name="s")
    f = pl.kernel(
        _k3_body,
        out_type=jax.ShapeDtypeStruct((B, 7 * ED), jnp.float32),
        mesh=mesh,
        compiler_params=pltpu.CompilerParams(needs_layout_passes=False, use_tc_tiling_on_sc=False),
        scratch_types=[
            pltpu.VMEM((16, 16), jnp.int32),
            pltpu.VMEM((NREL, 2 * ED), jnp.float32),
            pltpu.VMEM((NREL, 2 * ED), jnp.float32),
            pltpu.VMEM((NREL, 16, 16), jnp.int32),
            pltpu.VMEM((8, 3 * ED), jnp.float32),
            pltpu.VMEM((12, 16, 3 * ED), jnp.float32),
            pltpu.VMEM((12, 16), jnp.float32),
            pltpu.VMEM((8, 7 * ED), jnp.float32),
            pltpu.VMEM((N,), jnp.int32),
            pltpu.SemaphoreType.DMA,
            pltpu.SemaphoreType.DMA,
        ],
    )
    return f(h1, nodes2, adj2d, w2a, w2b)


def _k4_body(x_ref, w_ref, pg_ref, b_ref, out_ref):
    s = jnp.dot(x_ref[...], w_ref[...], preferred_element_type=jnp.float32)
    out_ref[...] = s + jnp.log(pg_ref[...]) + b_ref[...][0:1, :]


def _run_k4(x, W_out_pad, pg_pad, bq):
    return pl.pallas_call(
        _k4_body,
        out_shape=jax.ShapeDtypeStruct((B, 128), jnp.float32),
    )(x, W_out_pad, pg_pad, bq)


def kernel(nodes, feat_data, adj_lists, prior, W0, b0, alpha1, alpha2,
           W_out, b_out):
    nodes2 = nodes.reshape(16, 16)
    adj2d = adj_lists.reshape(NREL * N, K)

    a1t8 = jnp.zeros((8, 2 * ED * 2), jnp.float32)
    a1t8 = a1t8.at[0:NREL, 0:2 * ED].set(alpha1.T)
    a2t8 = jnp.zeros((8, 2 * ED * 2), jnp.float32)
    a2t8 = a2t8.at[4:4 + NREL, :].set(alpha2.T)

    h0w, wcat = _run_k1(feat_data, W0, b0, a1t8, a2t8)
    h0 = _run_k15(h0w)
    w1a = wcat[0:NREL, 0:ED]
    w1b = wcat[0:NREL, ED:2 * ED]
    w2a = wcat[4:4 + NREL, 0:2 * ED]
    w2b = wcat[4:4 + NREL, 2 * ED:]

    h1 = _run_k2(h0, nodes2, adj2d, w1a, w1b)
    x = _run_k3(h1, nodes2, adj2d, w2a, w2b)

    pg_pad = jnp.concatenate(
        [prior[nodes], jnp.ones((B, 126), jnp.float32)], axis=1
    )
    bq = jnp.broadcast_to(jnp.pad(b_out, (0, 126))[None, :], (8, 128))
    out = _run_k4(x, jnp.pad(W_out, ((0, 0), (0, 126))), pg_pad, bq)
    return out[:, :2]


# speedup vs baseline: 8.1736x; 1.0022x over previous
import functools

import jax
import jax.numpy as jnp
from jax import lax
from jax.experimental import pallas as pl
from jax.experimental.pallas import tpu as pltpu
from jax.experimental.pallas import tpu_sc as plsc

N = 50000
D = 128
K = 16
ED = 64
B = 256
NREL = 3
SL = B + NREL * B * K
SLP = SL + 8
NT = 32
NGRP = SL // K
GPT = (NGRP + NT - 1) // NT

BLK1 = 512
NPH = 49 * BLK1
NP = 2 * NPH
ZROW = 2 * (N - NPH) + 1


def _k1_body(fa_ref, fb_ref, w0_ref, b0_ref, a1_ref, a2_ref, h0_ref, w_ref):
    pid = pl.program_id(0)
    ha = jnp.dot(fa_ref[...], w0_ref[...], preferred_element_type=jnp.float32)
    hb = jnp.dot(fb_ref[...], w0_ref[...], preferred_element_type=jnp.float32)
    bias = b0_ref[...][0:1, :]
    rowb = NPH + pid * BLK1 + lax.broadcasted_iota(jnp.int32, (BLK1, ED), 0)
    hb = jnp.where(rowb < N, hb + bias, 0.0)
    h0_ref[...] = jnp.concatenate([ha + bias, hb], axis=1)

    @pl.when(pid == 0)
    def _():
        rid1 = lax.broadcasted_iota(jnp.int32, (8, 2 * ED * 2), 0)
        a1 = a1_ref[...]
        v1 = rid1 < NREL
        m1 = jnp.max(jnp.where(v1, a1, -jnp.inf), axis=0, keepdims=True)
        e1 = jnp.where(v1, jnp.exp(a1 - m1), 0.0)
        w1 = e1 / jnp.sum(e1, axis=0, keepdims=True)
        a2 = a2_ref[...]
        v2 = (rid1 >= 4) & (rid1 < 4 + NREL)
        m2 = jnp.max(jnp.where(v2, a2, -jnp.inf), axis=0, keepdims=True)
        e2 = jnp.where(v2, jnp.exp(a2 - m2), 0.0)
        w2 = e2 / jnp.sum(e2, axis=0, keepdims=True)
        w_ref[...] = jnp.where(v1, w1, 0.0) + jnp.where(v2, w2, 0.0)


def _run_k1(feat_data, W0, b0, a1t8, a2t8):
    b0b = jnp.broadcast_to(b0[None, :], (8, ED))
    nb = NPH // BLK1
    return pl.pallas_call(
        _k1_body,
        grid=(nb,),
        in_specs=[
            pl.BlockSpec((BLK1, D), lambda i: (i, 0)),
            pl.BlockSpec((BLK1, D), lambda i: (i + NPH // BLK1, 0)),
            pl.BlockSpec((D, ED), lambda i: (0, 0)),
            pl.BlockSpec((8, ED), lambda i: (0, 0)),
            pl.BlockSpec((8, 2 * ED * 2), lambda i: (0, 0)),
            pl.BlockSpec((8, 2 * ED * 2), lambda i: (0, 0)),
        ],
        out_specs=[
            pl.BlockSpec((BLK1, 2 * ED), lambda i: (i, 0)),
            pl.BlockSpec((8, 2 * ED * 2), lambda i: (0, 0)),
        ],
        out_shape=[
            jax.ShapeDtypeStruct((NPH, 2 * ED), jnp.float32),
            jax.ShapeDtypeStruct((8, 2 * ED * 2), jnp.float32),
        ],
    )(feat_data, feat_data, W0, b0b, a1t8, a2t8)


def _k15_body(h0w_hbm, tab_hbm, inbuf, outbuf, unused_sem):
    wid = lax.axis_index("s") * 2 + lax.axis_index("c")
    CH = 196
    for chunk in range(4):
        a = wid * 784 + chunk * CH
        pltpu.sync_copy(h0w_hbm.at[pl.ds(a * 2 * ED, CH * 2 * ED)], inbuf)

        def row_body(j, _):
            for half in range(2):
                for q in range(2):
                    c0 = inbuf[pl.ds(j * 2 * ED + half * 64 + q * 32, 16)]
                    c1 = inbuf[pl.ds(j * 2 * ED + half * 64 + q * 32 + 16, 16)]
                    p = plsc.pack(c0, c1, format=plsc.PackFormat.INTERLEAVED)
                    outbuf[2 * j + half, pl.ds(q * 32, 32)] = p
            return _

        lax.fori_loop(0, CH, row_body, None)
        pltpu.sync_copy(outbuf, tab_hbm.at[pl.ds(2 * a, 2 * CH)])


def _run_k15(h0w):
    mesh = plsc.VectorSubcoreMesh(core_axis_name="c", subcore_axis_name="s")
    f = pl.kernel(
        _k15_body,
        out_type=jax.ShapeDtypeStruct((NP, ED), jnp.bfloat16),
        mesh=mesh,
        compiler_params=pltpu.CompilerParams(
            needs_layout_passes=False, use_tc_tiling_on_sc=False),
        scratch_types=[
            pltpu.VMEM((196 * 2 * ED,), jnp.float32),
            pltpu.VMEM((392, ED), jnp.bfloat16),
            pltpu.SemaphoreType.DMA,
        ],
    )
    return f(h0w.reshape(-1))


def _dedup(neigh, iota, T):
    plsc.store_scatter(T, [neigh], iota)
    g = plsc.load_gather(T, [neigh])
    win = g == iota
    cnt = plsc.all_reduce_population_count(win)
    cinv = 1.0 / cnt.astype(jnp.float32)
    return win, cinv


def _k2_body(h0_hbm, nodes2_hbm, adj2d_hbm, w1a_hbm, w1b_hbm, h1_hbm,
             sids2, idxbuf, w1a_v, w1b_v, adjbuf, selfbuf, nfbuf, idxnf,
             cinvbuf, outbuf, T, semA, semS, semN, semO):
    wid = lax.axis_index("s") * 2 + lax.axis_index("c")
    iota = lax.iota(jnp.int32, 16)

    pltpu.sync_copy(nodes2_hbm, sids2.at[pl.ds(0, 16)])
    dStage = []
    for r in range(NREL):
        def _mk_idx(q, _):
            idxbuf[pl.ds(r * 256 + q * 16, 16)] = sids2[q, :] + r * N
            return _
        lax.fori_loop(0, 16, _mk_idx, None)
        dStage.append(pltpu.async_copy(
            adj2d_hbm.at[idxbuf.at[pl.ds(r * 256, 256)]],
            sids2.at[pl.ds(16 + r * 256, 256)], semA
        ))
    pltpu.sync_copy(w1a_hbm, w1a_v)
    pltpu.sync_copy(w1b_hbm, w1b_v)
    for d in dStage:
        d.wait()

    def pairmap(n):
        return jnp.where(n < NPH, 2 * n, 2 * n - (2 * NPH - 1))

    def fire_group(gg, sl):
        ids_vec = sids2[gg, :]
        pltpu.async_copy(h0_hbm.at[pairmap(ids_vec)], selfbuf.at[sl], semS)
        for r in range(NREL):
            pltpu.async_copy(
                adj2d_hbm.at[ids_vec + r * N], adjbuf.at[sl, r], semA
            )

    @pl.when(wid < NGRP)
    def _():
        fire_group(wid, 0)

    def group_body(it, _):
        gg = wid + it * NT
        sl = it % 2

        @pl.when(gg < NGRP)
        def _():
            for r in range(NREL):
                pltpu.make_async_copy(
                    adj2d_hbm.at[iota], adjbuf.at[sl, r], semA
                ).wait()

            def fire_half(h):
                def prep_row(j2, _):
                    for r in range(NREL):
                        neigh = adjbuf[sl, r, h * 8 + j2, :]
                        win, cinv = _dedup(neigh, iota, T)
                        idx2 = jnp.where(win, pairmap(neigh),
                                         jnp.full((16,), ZROW, jnp.int32))
                        slot = j2 * NREL + r
                        idxnf[h, pl.ds(slot * 16, 16)] = idx2
                        cinvbuf[h * 24 + slot, :] = cinv
                    return _
                lax.fori_loop(0, 8, prep_row, None)
                for d in range(3):
                    pltpu.async_copy(
                        h0_hbm.at[idxnf.at[h, pl.ds(d * 128, 128)]],
                        nfbuf.at[h, pl.ds(d * 128, 128)], semN
                    )

            def drain_half(h):
                for d in range(3):
                    pltpu.make_async_copy(
                        h0_hbm.at[idxnf.at[h, pl.ds(d * 128, 128)]],
                        nfbuf.at[h, pl.ds(d * 128, 128)], semN
                    ).wait()

            def unpack_row(ref, row):
                out = []
                for half2 in range(2):
                    v = ref[row, pl.ds(half2 * 32, 32)]
                    a, b = plsc.unpack(v, format=plsc.PackFormat.INTERLEAVED)
                    out.append(a)
                    out.append(b)
                return out

            def acc_half(h):
                def acc_row(j2, _):
                    sv = unpack_row(selfbuf.at[sl], h * 8 + j2)
                    i1 = [jnp.zeros((16,), jnp.float32) for _ in range(4)]
                    i2 = [jnp.zeros((16,), jnp.float32) for _ in range(4)]
                    for r in range(NREL):
                        slot = j2 * NREL + r
                        ci = cinvbuf[h * 24 + slot, :]
                        base = slot * 16
                        acc = [jnp.zeros((16,), jnp.float32) for _ in range(4)]
                        for k in range(16):
                            row = unpack_row(nfbuf.at[h], base + k)
                            for c in range(4):
                                acc[c] = acc[c] + row[c]
                        for c in range(4):
                            f1 = acc[c] * ci
                            wa = w1a_v[r, pl.ds(c * 16, 16)]
                            wb = w1b_v[r, pl.ds(c * 16, 16)]
                            i1[c] = i1[c] + f1 * wa
                            i2[c] = i2[c] + (sv[c] - f1) * wb
                    j = h * 8 + j2
                    for c in range(4):
                        outbuf[sl, j, pl.ds(c * 16, 16)] = sv[c]
                        outbuf[sl, j, pl.ds(ED + c * 16, 16)] = i1[c]
                        outbuf[sl, j, pl.ds(2 * ED + c * 16, 16)] = i2[c]
                    return _
                lax.fori_loop(0, 8, acc_row, None)

            fire_half(0)
            fire_half(1)
            ggn = gg + NT
            @pl.when(ggn < NGRP)
            def _():
                fire_group(ggn, (it + 1) % 2)
            pltpu.make_async_copy(h0_hbm.at[iota], selfbuf.at[sl], semS).wait()
            drain_half(0)
            acc_half(0)
            drain_half(1)
            acc_half(1)
            @pl.when(it >= 2)
            def _():
                pltpu.make_async_copy(
                    outbuf.at[sl], h1_hbm.at[pl.ds(0, 16)], semO
                ).wait()
            pltpu.async_copy(outbuf.at[sl], h1_hbm.at[pl.ds(gg * 16, 16)], semO)
        return _

    lax.fori_loop(0, GPT, group_body, None)

    for s in range(2):
        pltpu.make_async_copy(
            outbuf.at[s], h1_hbm.at[pl.ds(0, 16)], semO
        ).wait()

    @pl.when(wid == 0)
    def _():
        z = jnp.zeros((16,), jnp.float32)
        def zero_row(j, _):
            for c in range(12):
                outbuf[0, j, pl.ds(c * 16, 16)] = z
            return _
        lax.fori_loop(0, 8, zero_row, None)
        pltpu.sync_copy(outbuf.at[0, pl.ds(0, 8)], h1_hbm.at[pl.ds(SL, 8)])


def _run_k2(h0, nodes2, adj2d, w1a, w1b):
    mesh = plsc.VectorSubcoreMesh(core_axis_name="c", subcore_axis_name="s")
    f = pl.kernel(
        _k2_body,
        out_type=jax.ShapeDtypeStruct((SLP, 3 * ED), jnp.float32),
        mesh=mesh,
        compiler_params=pltpu.CompilerParams(needs_layout_passes=False, use_tc_tiling_on_sc=False),
        scratch_types=[
            pltpu.VMEM((NGRP, 16), jnp.int32),
            pltpu.VMEM((NREL * 256,), jnp.int32),
            pltpu.VMEM((NREL, ED), jnp.float32),
            pltpu.VMEM((NREL, ED), jnp.float32),
            pltpu.VMEM((2, NREL, 16, 16), jnp.int32),
            pltpu.VMEM((2, 16, ED), jnp.bfloat16),
            pltpu.VMEM((2, 384, ED), jnp.bfloat16),
            pltpu.VMEM((2, 384), jnp.int32),
            pltpu.VMEM((48, 16), jnp.float32),
            pltpu.VMEM((2, 16, 3 * ED), jnp.float32),
            pltpu.VMEM((N,), jnp.int32),
            pltpu.SemaphoreType.DMA,
            pltpu.SemaphoreType.DMA,
            pltpu.SemaphoreType.DMA,
            pltpu.SemaphoreType.DMA,
        ],
    )
    return f(h0, nodes2, adj2d, w1a, w1b)


def _k3_body(h1_hbm, nodes2_hbm, adj2d_hbm, w2a_hbm, w2b_hbm, x_hbm,
             nodes_v, w2a_v, w2b_v, adjbuf, selfbuf, nfbuf, cinvbuf,
             xbuf, T, semA, semN):
    wid = lax.axis_index("s") * 2 + lax.axis_index("c")
    iota = lax.iota(jnp.int32, 16)

    pltpu.sync_copy(nodes2_hbm, nodes_v)
    pltpu.sync_copy(w2a_hbm, w2a_v)
    pltpu.sync_copy(w2b_hbm, w2b_v)
    ids_vec = nodes_v[wid // 2, :]
    dA = [
        pltpu.async_copy(adj2d_hbm.at[ids_vec + r * N], adjbuf.at[r], semA)
        for r in range(NREL)
    ]
    pltpu.sync_copy(h1_hbm.at[pl.ds(wid * 8, 8)], selfbuf)
    for d in dA:
        d.wait()

    for half in range(2):
        def fire_node(l2, _):
            lane = (wid % 2) * 8 + half * 4 + l2
            i_node = wid * 8 + half * 4 + l2
            for r in range(NREL):
                neigh = adjbuf[r, lane, :]
                win, cinv = _dedup(neigh, iota, T)
                pos = B + r * B * K + i_node * K + iota
                idx2 = jnp.where(win, pos, jnp.full((16,), SL, jnp.int32))
                slot = l2 * NREL + r
                pltpu.async_copy(h1_hbm.at[idx2], nfbuf.at[slot], semN)
                cinvbuf[slot, :] = cinv
            return _

        lax.fori_loop(0, 4, fire_node, None)
        for s in range(12):
            pltpu.make_async_copy(h1_hbm.at[iota], nfbuf.at[s], semN).wait()

        def acc_node(l2, _):
            lrow = half * 4 + l2
            sf = [selfbuf[lrow, pl.ds(ED + c * 16, 16)] for c in range(8)]
            i1 = [jnp.zeros((16,), jnp.float32) for _ in range(8)]
            i2 = [jnp.zeros((16,), jnp.float32) for _ in range(8)]
            for r in range(NREL):
                slot = l2 * NREL + r
                ci = cinvbuf[slot, :]
                for c in range(8):
                    a = nfbuf[slot, 0, pl.ds(ED + c * 16, 16)]
                    for k in range(1, 16):
                        a = a + nfbuf[slot, k, pl.ds(ED + c * 16, 16)]
                    f1 = a * ci
                    wa = w2a_v[r, pl.ds(c * 16, 16)]
                    wb = w2b_v[r, pl.ds(c * 16, 16)]
                    i1[c] = i1[c] + f1 * wa
                    i2[c] = i2[c] + (sf[c] - f1) * wb
            for c in range(12):
                xbuf[lrow, pl.ds(c * 16, 16)] = selfbuf[lrow, pl.ds(c * 16, 16)]
            for c in range(8):
                xbuf[lrow, pl.ds(3 * ED + c * 16, 16)] = i1[c]
                xbuf[lrow, pl.ds(5 * ED + c * 16, 16)] = i2[c]
            return _

        lax.fori_loop(0, 4, acc_node, None)
    pltpu.sync_copy(xbuf, x_hbm.at[pl.ds(wid * 8, 8)])


def _run_k3(h1, nodes2, adj2d, w2a, w2b):
    mesh = plsc.VectorSubcoreMesh(core_axis_name="c", subcore_axis_name="s")
    f = pl.kernel(
        _k3_body,
        out_type=jax.ShapeDtypeStruct((B, 7 * ED), jnp.float32),
        mesh=mesh,
        compiler_params=pltpu.CompilerParams(needs_layout_passes=False, use_tc_tiling_on_sc=False),
        scratch_types=[
            pltpu.VMEM((16, 16), jnp.int32),
            pltpu.VMEM((NREL, 2 * ED), jnp.float32),
            pltpu.VMEM((NREL, 2 * ED), jnp.float32),
            pltpu.VMEM((NREL, 16, 16), jnp.int32),
            pltpu.VMEM((8, 3 * ED), jnp.float32),
            pltpu.VMEM((12, 16, 3 * ED), jnp.float32),
            pltpu.VMEM((12, 16), jnp.float32),
            pltpu.VMEM((8, 7 * ED), jnp.float32),
            pltpu.VMEM((N,), jnp.int32),
            pltpu.SemaphoreType.DMA,
            pltpu.SemaphoreType.DMA,
        ],
    )
    return f(h1, nodes2, adj2d, w2a, w2b)


def _k4_body(x_ref, w_ref, pg_ref, b_ref, out_ref):
    s = jnp.dot(x_ref[...], w_ref[...], preferred_element_type=jnp.float32)
    out_ref[...] = s + jnp.log(pg_ref[...]) + b_ref[...][0:1, :]


def _run_k4(x, W_out_pad, pg_pad, bq):
    return pl.pallas_call(
        _k4_body,
        out_shape=jax.ShapeDtypeStruct((B, 128), jnp.float32),
    )(x, W_out_pad, pg_pad, bq)


def kernel(nodes, feat_data, adj_lists, prior, W0, b0, alpha1, alpha2,
           W_out, b_out):
    nodes2 = nodes.reshape(16, 16)
    adj2d = adj_lists.reshape(NREL * N, K)

    a1t8 = jnp.zeros((8, 2 * ED * 2), jnp.float32)
    a1t8 = a1t8.at[0:NREL, 0:2 * ED].set(alpha1.T)
    a2t8 = jnp.zeros((8, 2 * ED * 2), jnp.float32)
    a2t8 = a2t8.at[4:4 + NREL, :].set(alpha2.T)

    h0w, wcat = _run_k1(feat_data, W0, b0, a1t8, a2t8)
    h0 = _run_k15(h0w)
    w1a = wcat[0:NREL, 0:ED]
    w1b = wcat[0:NREL, ED:2 * ED]
    w2a = wcat[4:4 + NREL, 0:2 * ED]
    w2b = wcat[4:4 + NREL, 2 * ED:]

    h1 = _run_k2(h0, nodes2, adj2d, w1a, w1b)
    x = _run_k3(h1, nodes2, adj2d, w2a, w2b)

    pg_pad = jnp.concatenate(
        [prior[nodes], jnp.ones((B, 126), jnp.float32)], axis=1
    )
    bq = jnp.broadcast_to(jnp.pad(b_out, (0, 126))[None, :], (8, 128))
    out = _run_k4(x, jnp.pad(W_out, ((0, 0), (0, 126))), pg_pad, bq)
    return out[:, :2]
